# Initial kernel scaffold; baseline (speedup 1.0000x reference)
import jax
import jax.numpy as jnp
from jax.experimental import pallas as pl


def kernel(train_ids, node2edge_idx, edge_node_adj, feats, edge_emb, W_prep0, W_prep1, W_ep, W_e0, W_e1, W_n0, W_n1, a_mp, W_fc, b_fc):
    raise NotImplementedError("write your pallas kernel here")


# speedup vs baseline: 5.3735x; 5.3735x over previous
import functools

import jax
import jax.numpy as jnp
from jax import lax
from jax.experimental import pallas as pl
from jax.experimental.pallas import tpu as pltpu
from jax.experimental.pallas import tpu_sc as plsc

N = 10000
S = 32
E = 320000
D = 128
ED = 16
H = 128
C = 16
B = 8192

NC = 2
NS = 16
NWORK = NC * NS

_HIGH = jax.lax.Precision.HIGHEST


def _vmesh():
    return plsc.VectorSubcoreMesh(core_axis_name="c", subcore_axis_name="s")


_SC_PARAMS = pltpu.CompilerParams(use_tc_tiling_on_sc=False)


def _pair_sum(g, a0, a1):
    CH = 80
    per_w = E // NWORK
    n_ch = per_w // CH

    @functools.partial(
        pl.kernel,
        out_type=jax.ShapeDtypeStruct((E, H), jnp.float32),
        mesh=_vmesh(),
        compiler_params=_SC_PARAMS,
        scratch_types=[
            pltpu.VMEM((CH,), jnp.int32),
            pltpu.VMEM((CH,), jnp.int32),
            pltpu.VMEM((CH, H), jnp.float32),
            pltpu.VMEM((CH, H), jnp.float32),
            pltpu.SemaphoreType.DMA,
            pltpu.SemaphoreType.DMA,
        ],
    )
    def k(g_hbm, a0_hbm, a1_hbm, o_hbm, i0_v, i1_v, r0_v, r1_v, sem0, sem1):
        wid = lax.axis_index("s") * NC + lax.axis_index("c")
        base = wid * per_w

        @pl.loop(0, n_ch)
        def _(j):
            off = base + j * CH
            pltpu.sync_copy(a0_hbm.at[pl.ds(off, CH)], i0_v)
            pltpu.sync_copy(a1_hbm.at[pl.ds(off, CH)], i1_v)
            cp0 = pltpu.async_copy(g_hbm.at[i0_v], r0_v, sem0)
            cp1 = pltpu.async_copy(g_hbm.at[i1_v], r1_v, sem1)
            cp0.wait()
            cp1.wait()

            @pl.loop(0, CH)
            def _(r):
                for cc in range(H // 16):
                    sl = pl.ds(cc * 16, 16)
                    r0_v[r, sl] = r0_v[r, sl] + r1_v[r, sl]

            pltpu.sync_copy(r0_v, o_hbm.at[pl.ds(off, CH)])

    return k(g, a0, a1)


def _tid_gather(n2e, p, tids):
    CH = 128
    per_w = B // NWORK
    n_ch = per_w // CH

    @functools.partial(
        pl.kernel,
        out_type=(jax.ShapeDtypeStruct((B, S), jnp.int32),
                  jax.ShapeDtypeStruct((B, H), jnp.float32)),
        mesh=_vmesh(),
        compiler_params=_SC_PARAMS,
        scratch_types=[
            pltpu.VMEM((CH,), jnp.int32),
            pltpu.VMEM((CH, S), jnp.int32),
            pltpu.VMEM((CH, H), jnp.float32),
            pltpu.SemaphoreType.DMA,
            pltpu.SemaphoreType.DMA,
        ],
    )
    def k(n2e_hbm, p_hbm, t_hbm, o1_hbm, o2_hbm, i_v, r1_v, r2_v, s1, s2):
        wid = lax.axis_index("s") * NC + lax.axis_index("c")
        base = wid * per_w

        @pl.loop(0, n_ch)
        def _(j):
            off = base + j * CH
            pltpu.sync_copy(t_hbm.at[pl.ds(off, CH)], i_v)
            c1 = pltpu.async_copy(n2e_hbm.at[i_v], r1_v, s1)
            c2 = pltpu.async_copy(p_hbm.at[i_v], r2_v, s2)
            c1.wait()
            c2.wait()
            pltpu.sync_copy(r1_v, o1_hbm.at[pl.ds(off, CH)])
            pltpu.sync_copy(r2_v, o2_hbm.at[pl.ds(off, CH)])

    return k(n2e, p, tids)


def _seg_sum(data, idx_flat, dim):
    NPC = 8
    CHI = NPC * S // 2
    per_w = B // NWORK
    n_ch = per_w // NPC

    @functools.partial(
        pl.kernel,
        out_type=jax.ShapeDtypeStruct((B, dim), jnp.float32),
        mesh=_vmesh(),
        compiler_params=_SC_PARAMS,
        scratch_types=[
            pltpu.VMEM((CHI,), jnp.int32),
            pltpu.VMEM((CHI,), jnp.int32),
            pltpu.VMEM((CHI, dim), jnp.float32),
            pltpu.VMEM((CHI, dim), jnp.float32),
            pltpu.VMEM((NPC, dim), jnp.float32),
            pltpu.SemaphoreType.DMA,
            pltpu.SemaphoreType.DMA,
        ],
    )
    def k(d_hbm, i_hbm, o_hbm, i0_v, i1_v, r0_v, r1_v, acc_v, s0, s1):
        wid = lax.axis_index("s") * NC + lax.axis_index("c")
        base = wid * per_w

        @pl.loop(0, n_ch)
        def _(j):
            node0 = base + j * NPC
            pltpu.sync_copy(i_hbm.at[pl.ds(node0 * S, CHI)], i0_v)
            pltpu.sync_copy(i_hbm.at[pl.ds(node0 * S + CHI, CHI)], i1_v)
            c0 = pltpu.async_copy(d_hbm.at[i0_v], r0_v, s0)
            c1 = pltpu.async_copy(d_hbm.at[i1_v], r1_v, s1)
            c0.wait()
            c1.wait()

            @pl.loop(0, NPC // 2)
            def _(n):
                rbase = n * S
                for buf, arow in ((r0_v, 0), (r1_v, NPC // 2)):
                    for cc in range(dim // 16):
                        sl = pl.ds(cc * 16, 16)
                        vals = [buf[rbase + s, sl] for s in range(S)]
                        while len(vals) > 1:
                            vals = [vals[i] + vals[i + 1]
                                    for i in range(0, len(vals) - 1, 2)] + (
                                        [vals[-1]] if len(vals) % 2 else [])
                        acc_v[arow + n, sl] = vals[0]

            pltpu.sync_copy(acc_v, o_hbm.at[pl.ds(node0, NPC)])

    return k(data, idx_flat)


def _prep(f, wg, wp):
    BK = 1000

    def body(f_ref, wg_ref, wp_ref, g_ref, p_ref):
        x = f_ref[...]
        g_ref[...] = jnp.dot(x, wg_ref[...], precision=_HIGH)
        p_ref[...] = jnp.dot(x, wp_ref[...], precision=_HIGH)

    return pl.pallas_call(
        body,
        grid=(N // BK,),
        in_specs=[
            pl.BlockSpec((BK, D), lambda i: (i, 0)),
            pl.BlockSpec((D, H), lambda i: (0, 0)),
            pl.BlockSpec((D, H), lambda i: (0, 0)),
        ],
        out_specs=[
            pl.BlockSpec((BK, H), lambda i: (i, 0)),
            pl.BlockSpec((BK, H), lambda i: (i, 0)),
        ],
        out_shape=[jax.ShapeDtypeStruct((N, H), jnp.float32)] * 2,
    )(f, wg, wp)


def _edges(psum, emb, wc1):
    BE = 4000

    def body(ps_ref, e_ref, w_ref, z_ref):
        m = jnp.dot(e_ref[...], w_ref[...], precision=_HIGH)
        z_ref[...] = jnp.maximum(ps_ref[...] + m, 0.0)

    return pl.pallas_call(
        body,
        grid=(E // BE,),
        in_specs=[
            pl.BlockSpec((BE, H), lambda i: (i, 0)),
            pl.BlockSpec((BE, ED), lambda i: (i, 0)),
            pl.BlockSpec((ED, H), lambda i: (0, 0)),
        ],
        out_specs=pl.BlockSpec((BE, H), lambda i: (i, 0)),
        out_shape=jax.ShapeDtypeStruct((E, H), jnp.float32),
    )(psum, emb, wc1)


def _head(p_t, s0, e1, b0, wn1a, wn1b, wfc0, wfc1, bfc):
    BB = 1024

    def body(p_ref, s0_ref, e1_ref, b0_ref, wa_ref, wb_ref, w0_ref, w1_ref,
             bf_ref, lg_ref, wt_ref):
        f0 = jnp.maximum(
            p_ref[...] + jnp.dot(s0_ref[...], b0_ref[...], precision=_HIGH),
            0.0)
        f1 = jnp.maximum(
            jnp.dot(f0, wa_ref[...], precision=_HIGH)
            + jnp.dot(e1_ref[...], wb_ref[...], precision=_HIGH), 0.0)
        lg_ref[...] = (jnp.dot(f0, w0_ref[...], precision=_HIGH)
                       + jnp.dot(f1, w1_ref[...], precision=_HIGH)
                       + bf_ref[...])
        wt_ref[...] = jnp.ones((1, BB), jnp.float32)

    return pl.pallas_call(
        body,
        grid=(B // BB,),
        in_specs=[
            pl.BlockSpec((BB, H), lambda i: (i, 0)),
            pl.BlockSpec((BB, ED), lambda i: (i, 0)),
            pl.BlockSpec((BB, H), lambda i: (i, 0)),
            pl.BlockSpec((ED, H), lambda i: (0, 0)),
            pl.BlockSpec((H, H), lambda i: (0, 0)),
            pl.BlockSpec((H, H), lambda i: (0, 0)),
            pl.BlockSpec((H, C), lambda i: (0, 0)),
            pl.BlockSpec((H, C), lambda i: (0, 0)),
            pl.BlockSpec((1, C), lambda i: (0, 0)),
        ],
        out_specs=[
            pl.BlockSpec((BB, C), lambda i: (i, 0)),
            pl.BlockSpec((1, BB), lambda i: (0, i)),
        ],
        out_shape=[
            jax.ShapeDtypeStruct((B, C), jnp.float32),
            jax.ShapeDtypeStruct((1, B), jnp.float32),
        ],
    )(p_t, s0, e1, b0, wn1a, wn1b, wfc0, wfc1, bfc)


def kernel(train_ids, node2edge_idx, edge_node_adj, feats, edge_emb,
           W_prep0, W_prep1, W_ep, W_e0, W_e1, W_n0, W_n1, a_mp, W_fc, b_fc):
    wc1 = W_ep @ W_e0[:H]
    wg = 0.5 * (W_prep1 @ W_e0[H:])
    wp = W_prep0 @ W_n0[:H]
    b0 = (W_ep @ W_n0[H:]) / S
    wn1a = W_n1[:H]
    wn1b = W_n1[H:] / S
    wfc0 = W_fc[:H]
    wfc1 = W_fc[H:]
    bfc = b_fc.reshape(1, C)

    a0 = edge_node_adj[:, 0]
    a1 = edge_node_adj[:, 1]
    tids = train_ids.astype(jnp.int32)

    g, p = _prep(feats, wg, wp)
    psum = _pair_sum(g, a0, a1)
    z = _edges(psum, edge_emb, wc1)
    n2e_t, p_t = _tid_gather(node2edge_idx, p, tids)
    idx_flat = n2e_t.reshape(B * S)
    s0 = _seg_sum(edge_emb, idx_flat, ED)
    e1 = _seg_sum(z, idx_flat, H)
    logits, weights = _head(p_t, s0, e1, b0, wn1a, wn1b, wfc0, wfc1, bfc)
    return (logits, weights)


# speedup vs baseline: 8.0815x; 1.5039x over previous
import functools

import jax
import jax.numpy as jnp
from jax import lax
from jax.experimental import pallas as pl
from jax.experimental.pallas import tpu as pltpu
from jax.experimental.pallas import tpu_sc as plsc

N = 10000
S = 32
E = 320000
D = 128
ED = 16
H = 128
C = 16
B = 8192

NC = 2
NS = 16
NWORK = NC * NS

_PREC = None


def _vmesh():
    return plsc.VectorSubcoreMesh(core_axis_name="c", subcore_axis_name="s")


_SC_PARAMS = pltpu.CompilerParams(use_tc_tiling_on_sc=False)


def _pair_sum(g, a0, a1):
    CH = 80
    per_w = E // NWORK
    n_ch = per_w // CH
    NSLOT = 5

    @functools.partial(
        pl.kernel,
        out_type=jax.ShapeDtypeStruct((E, H), jnp.float32),
        mesh=_vmesh(),
        compiler_params=_SC_PARAMS,
        scratch_types=(
            [pltpu.VMEM((per_w,), jnp.int32)] * 2
            + [pltpu.VMEM((CH, H), jnp.float32)] * (2 * NSLOT)
            + [pltpu.SemaphoreType.DMA] * (2 * NSLOT)
        ),
    )
    def k(g_hbm, a0_hbm, a1_hbm, o_hbm, *bufs):
        ia_v, ib_v = bufs[0], bufs[1]
        r0 = bufs[2:2 + NSLOT]
        r1 = bufs[2 + NSLOT:2 + 2 * NSLOT]
        gsem = bufs[2 + 2 * NSLOT:2 + 3 * NSLOT]
        osem = bufs[2 + 3 * NSLOT:2 + 4 * NSLOT]
        wid = lax.axis_index("s") * NC + lax.axis_index("c")
        base = wid * per_w
        pltpu.sync_copy(a0_hbm.at[pl.ds(base, per_w)], ia_v)
        pltpu.sync_copy(a1_hbm.at[pl.ds(base, per_w)], ib_v)

        def issue(kk, s):
            pltpu.async_copy(g_hbm.at[ia_v.at[pl.ds(kk * CH, CH)]],
                             r0[s], gsem[s])
            pltpu.async_copy(g_hbm.at[ib_v.at[pl.ds(kk * CH, CH)]],
                             r1[s], gsem[s])

        for s in range(3):
            issue(s, s)

        @pl.loop(0, n_ch // NSLOT)
        def _(k5):
            for b in range(NSLOT):
                kk = k5 * NSLOT + b
                s3 = (b + 3) % NSLOT

                @pl.when(kk >= 2)
                def _():
                    pltpu.make_async_copy(
                        r0[s3], o_hbm.at[pl.ds(0, CH)], osem[s3]).wait()

                @pl.when(kk < n_ch - 3)
                def _():
                    issue(kk + 3, s3)

                pltpu.make_async_copy(g_hbm.at[ia_v.at[pl.ds(0, CH)]],
                                      r0[b], gsem[b]).wait()
                pltpu.make_async_copy(g_hbm.at[ib_v.at[pl.ds(0, CH)]],
                                      r1[b], gsem[b]).wait()

                @pl.loop(0, CH)
                def _(r):
                    for cc in range(H // 16):
                        sl = pl.ds(cc * 16, 16)
                        r0[b][r, sl] = r0[b][r, sl] + r1[b][r, sl]

                pltpu.async_copy(
                    r0[b], o_hbm.at[pl.ds(base + kk * CH, CH)], osem[b])

        for kk in (n_ch - 2, n_ch - 1):
            s = kk % NSLOT
            pltpu.make_async_copy(
                r0[s], o_hbm.at[pl.ds(0, CH)], osem[s]).wait()

    return k(g, a0, a1)


def _tid_gather(n2e, p, tids):
    CH = 128
    per_w = B // NWORK
    n_ch = per_w // CH

    @functools.partial(
        pl.kernel,
        out_type=(jax.ShapeDtypeStruct((B, S), jnp.int32),
                  jax.ShapeDtypeStruct((B, H), jnp.float32)),
        mesh=_vmesh(),
        compiler_params=_SC_PARAMS,
        scratch_types=[
            pltpu.VMEM((CH,), jnp.int32),
            pltpu.VMEM((CH, S), jnp.int32),
            pltpu.VMEM((CH, H), jnp.float32),
            pltpu.SemaphoreType.DMA,
            pltpu.SemaphoreType.DMA,
        ],
    )
    def k(n2e_hbm, p_hbm, t_hbm, o1_hbm, o2_hbm, i_v, r1_v, r2_v, s1, s2):
        wid = lax.axis_index("s") * NC + lax.axis_index("c")
        base = wid * per_w

        @pl.loop(0, n_ch)
        def _(j):
            off = base + j * CH
            pltpu.sync_copy(t_hbm.at[pl.ds(off, CH)], i_v)
            c1 = pltpu.async_copy(n2e_hbm.at[i_v], r1_v, s1)
            c2 = pltpu.async_copy(p_hbm.at[i_v], r2_v, s2)
            c1.wait()
            c2.wait()
            pltpu.sync_copy(r1_v, o1_hbm.at[pl.ds(off, CH)])
            pltpu.sync_copy(r2_v, o2_hbm.at[pl.ds(off, CH)])

    return k(n2e, p, tids)


def _seg_sum(data, idx_flat, dim):
    NPC = 8
    CHI = NPC * S // 2
    per_w = B // NWORK
    n_ch = per_w // NPC

    @functools.partial(
        pl.kernel,
        out_type=jax.ShapeDtypeStruct((B, dim), jnp.float32),
        mesh=_vmesh(),
        compiler_params=_SC_PARAMS,
        scratch_types=(
            [pltpu.VMEM((per_w * S,), jnp.int32)]
            + [pltpu.VMEM((CHI, dim), jnp.float32)] * 4
            + [pltpu.VMEM((NPC, dim), jnp.float32)] * 2
            + [pltpu.SemaphoreType.DMA] * 4
        ),
    )
    def k(d_hbm, i_hbm, o_hbm, *bufs):
        i_v = bufs[0]
        r0 = bufs[1:3]
        r1 = bufs[3:5]
        acc = bufs[5:7]
        gsem = bufs[7:9]
        osem = bufs[9:11]
        wid = lax.axis_index("s") * NC + lax.axis_index("c")
        base = wid * per_w
        pltpu.sync_copy(i_hbm.at[pl.ds(base * S, per_w * S)], i_v)

        def issue(kk, s):
            pltpu.async_copy(
                d_hbm.at[i_v.at[pl.ds(kk * NPC * S, CHI)]], r0[s], gsem[s])
            pltpu.async_copy(
                d_hbm.at[i_v.at[pl.ds(kk * NPC * S + CHI, CHI)]],
                r1[s], gsem[s])

        issue(0, 0)

        @pl.loop(0, n_ch // 2)
        def _(k2):
            for b in range(2):
                kk = k2 * 2 + b

                @pl.when(kk < n_ch - 1)
                def _():
                    issue(kk + 1, 1 - b)

                pltpu.make_async_copy(d_hbm.at[i_v.at[pl.ds(0, CHI)]],
                                      r0[b], gsem[b]).wait()
                pltpu.make_async_copy(d_hbm.at[i_v.at[pl.ds(0, CHI)]],
                                      r1[b], gsem[b]).wait()

                @pl.when(kk >= 2)
                def _():
                    pltpu.make_async_copy(
                        acc[b], o_hbm.at[pl.ds(0, NPC)], osem[b]).wait()

                @pl.loop(0, NPC // 2)
                def _(n):
                    rbase = n * S
                    for buf, arow in ((r0[b], 0), (r1[b], NPC // 2)):
                        for cc in range(dim // 16):
                            sl = pl.ds(cc * 16, 16)
                            vals = [buf[rbase + s, sl] for s in range(S)]
                            while len(vals) > 1:
                                vals = [vals[i] + vals[i + 1]
                                        for i in range(0, len(vals) - 1, 2)] + (
                                            [vals[-1]] if len(vals) % 2 else [])
                            acc[b][arow + n, sl] = vals[0]

                pltpu.async_copy(
                    acc[b], o_hbm.at[pl.ds(base + kk * NPC, NPC)], osem[b])

        for kk in (n_ch - 2, n_ch - 1):
            pltpu.make_async_copy(
                acc[kk % 2], o_hbm.at[pl.ds(0, NPC)], osem[kk % 2]).wait()

    return k(data, idx_flat)


def _prep(f, wg, wp):
    BK = 1000

    def body(f_ref, wg_ref, wp_ref, g_ref, p_ref):
        x = f_ref[...]
        g_ref[...] = jnp.dot(x, wg_ref[...], precision=_PREC)
        p_ref[...] = jnp.dot(x, wp_ref[...], precision=_PREC)

    return pl.pallas_call(
        body,
        grid=(N // BK,),
        in_specs=[
            pl.BlockSpec((BK, D), lambda i: (i, 0)),
            pl.BlockSpec((D, H), lambda i: (0, 0)),
            pl.BlockSpec((D, H), lambda i: (0, 0)),
        ],
        out_specs=[
            pl.BlockSpec((BK, H), lambda i: (i, 0)),
            pl.BlockSpec((BK, H), lambda i: (i, 0)),
        ],
        out_shape=[jax.ShapeDtypeStruct((N, H), jnp.float32)] * 2,
    )(f, wg, wp)


def _edges(psum, emb, wc1):
    BE = 4000

    def body(ps_ref, e_ref, w_ref, z_ref):
        m = jnp.dot(e_ref[...], w_ref[...], precision=_PREC)
        z_ref[...] = jnp.maximum(ps_ref[...] + m, 0.0)

    return pl.pallas_call(
        body,
        grid=(E // BE,),
        in_specs=[
            pl.BlockSpec((BE, H), lambda i: (i, 0)),
            pl.BlockSpec((BE, ED), lambda i: (i, 0)),
            pl.BlockSpec((ED, H), lambda i: (0, 0)),
        ],
        out_specs=pl.BlockSpec((BE, H), lambda i: (i, 0)),
        out_shape=jax.ShapeDtypeStruct((E, H), jnp.float32),
    )(psum, emb, wc1)


def _head(p_t, s0, e1, b0, wn1a, wn1b, wfc0, wfc1, bfc):
    BB = 1024

    def body(p_ref, s0_ref, e1_ref, b0_ref, wa_ref, wb_ref, w0_ref, w1_ref,
             bf_ref, lg_ref, wt_ref):
        f0 = jnp.maximum(
            p_ref[...] + jnp.dot(s0_ref[...], b0_ref[...], precision=_PREC),
            0.0)
        f1 = jnp.maximum(
            jnp.dot(f0, wa_ref[...], precision=_PREC)
            + jnp.dot(e1_ref[...], wb_ref[...], precision=_PREC), 0.0)
        lg_ref[...] = (jnp.dot(f0, w0_ref[...], precision=_PREC)
                       + jnp.dot(f1, w1_ref[...], precision=_PREC)
                       + bf_ref[...])
        wt_ref[...] = jnp.ones((1, BB), jnp.float32)

    return pl.pallas_call(
        body,
        grid=(B // BB,),
        in_specs=[
            pl.BlockSpec((BB, H), lambda i: (i, 0)),
            pl.BlockSpec((BB, ED), lambda i: (i, 0)),
            pl.BlockSpec((BB, H), lambda i: (i, 0)),
            pl.BlockSpec((ED, H), lambda i: (0, 0)),
            pl.BlockSpec((H, H), lambda i: (0, 0)),
            pl.BlockSpec((H, H), lambda i: (0, 0)),
            pl.BlockSpec((H, C), lambda i: (0, 0)),
            pl.BlockSpec((H, C), lambda i: (0, 0)),
            pl.BlockSpec((1, C), lambda i: (0, 0)),
        ],
        out_specs=[
            pl.BlockSpec((BB, C), lambda i: (i, 0)),
            pl.BlockSpec((1, BB), lambda i: (0, i)),
        ],
        out_shape=[
            jax.ShapeDtypeStruct((B, C), jnp.float32),
            jax.ShapeDtypeStruct((1, B), jnp.float32),
        ],
    )(p_t, s0, e1, b0, wn1a, wn1b, wfc0, wfc1, bfc)


def kernel(train_ids, node2edge_idx, edge_node_adj, feats, edge_emb,
           W_prep0, W_prep1, W_ep, W_e0, W_e1, W_n0, W_n1, a_mp, W_fc, b_fc):
    wc1 = W_ep @ W_e0[:H]
    wg = 0.5 * (W_prep1 @ W_e0[H:])
    wp = W_prep0 @ W_n0[:H]
    b0 = (W_ep @ W_n0[H:]) / S
    wn1a = W_n1[:H]
    wn1b = W_n1[H:] / S
    wfc0 = W_fc[:H]
    wfc1 = W_fc[H:]
    bfc = b_fc.reshape(1, C)

    a0 = edge_node_adj[:, 0]
    a1 = edge_node_adj[:, 1]
    tids = train_ids.astype(jnp.int32)

    g, p = _prep(feats, wg, wp)
    psum = _pair_sum(g, a0, a1)
    z = _edges(psum, edge_emb, wc1)
    n2e_t, p_t = _tid_gather(node2edge_idx, p, tids)
    idx_flat = n2e_t.reshape(B * S)
    s0 = _seg_sum(edge_emb, idx_flat, ED)
    e1 = _seg_sum(z, idx_flat, H)
    logits, weights = _head(p_t, s0, e1, b0, wn1a, wn1b, wfc0, wfc1, bfc)
    return (logits, weights)
